# baseline (speedup 1.0000x reference)
import functools

import jax
import jax.numpy as jnp
from jax.experimental import pallas as pl
from jax.experimental.pallas import tpu as pltpu

DIM = 768
N_EXPERTS = 64
TOP_K = 8
TOKENS = 32768
BLOCK = 8192
NCHUNK = 64


def _router_kernel(x_ref, w_ref, b_ref, w_out, i_out, loss_out,
                   counts_ref, probsum_ref):
    step = pl.program_id(0)
    nsteps = pl.num_programs(0)

    @pl.when(step == 0)
    def _init():
        counts_ref[...] = jnp.zeros_like(counts_ref)
        probsum_ref[...] = jnp.zeros_like(probsum_ref)

    lt = jax.lax.dot_general(
        w_ref[...], x_ref[...],
        dimension_numbers=(((0,), (1,)), ((), ())),
        preferred_element_type=jnp.float32) + b_ref[...]
    s = jax.nn.sigmoid(lt)
    d = jnp.sum(s, axis=0, keepdims=True) + 1e-6
    rd = 1.0 / d
    probsum_ref[...] += jax.lax.dot_general(
        s, rd, dimension_numbers=(((1,), (1,)), ((), ())),
        preferred_element_type=jnp.float32)

    csz = BLOCK // NCHUNK
    eidx = jax.lax.broadcasted_iota(jnp.int32, (N_EXPERTS, csz), 0)
    works = [s[:, c * csz:(c + 1) * csz] for c in range(NCHUNK)]
    ws = [[] for _ in range(NCHUNK)]
    idxs = [[] for _ in range(NCHUNK)]
    for _ in range(TOP_K):
        for c in range(NCHUNK):
            v, i = works[c], eidx
            half = N_EXPERTS // 2
            while half >= 1:
                va, vb = v[:half], v[half:]
                ia, ib = i[:half], i[half:]
                keep = va >= vb
                v = jnp.maximum(va, vb)
                i = jnp.where(keep, ia, ib)
                half //= 2
            first_b = eidx == i
            ws[c].append(v)
            idxs[c].append(i)
            works[c] = jnp.where(first_b, -1.0, works[c])

    counts = jnp.zeros((N_EXPERTS, 1), jnp.float32)
    for c in range(NCHUNK):
        neg = (works[c] < 0.0).astype(jnp.float32)
        counts = counts + jnp.sum(neg, axis=1, keepdims=True)
    counts_ref[...] += counts

    wstack = jnp.concatenate(
        [jnp.concatenate(ws[c], axis=0) for c in range(NCHUNK)],
        axis=1)
    wstack = wstack / (jnp.sum(wstack, axis=0, keepdims=True) + 1e-6 * d)
    istack = jnp.concatenate(
        [jnp.concatenate(idxs[c], axis=0) for c in range(NCHUNK)],
        axis=1)

    w_out[...] = wstack
    i_out[...] = istack

    @pl.when(step == nsteps - 1)
    def _finalize():
        scale = N_EXPERTS / (TOKENS * TOKENS)
        loss = scale * jnp.sum(counts_ref[...] * probsum_ref[...])
        loss_out[...] = jnp.full((1, 1), loss, dtype=jnp.float32)


@functools.partial(jax.jit, static_argnames=())
def kernel(x, W, expert_bias):
    grid = TOKENS // BLOCK
    weights, indices, loss = pl.pallas_call(
        _router_kernel,
        grid=(grid,),
        in_specs=[
            pl.BlockSpec((BLOCK, DIM), lambda i: (i, 0)),
            pl.BlockSpec((DIM, N_EXPERTS), lambda i: (0, 0)),
            pl.BlockSpec((N_EXPERTS, 1), lambda i: (0, 0)),
        ],
        out_specs=[
            pl.BlockSpec((TOP_K, BLOCK), lambda i: (0, i)),
            pl.BlockSpec((TOP_K, BLOCK), lambda i: (0, i)),
            pl.BlockSpec((1, 1), lambda i: (0, 0)),
        ],
        out_shape=[
            jax.ShapeDtypeStruct((TOP_K, TOKENS), jnp.float32),
            jax.ShapeDtypeStruct((TOP_K, TOKENS), jnp.int32),
            jax.ShapeDtypeStruct((1, 1), jnp.float32),
        ],
        scratch_shapes=[
            pltpu.VMEM((N_EXPERTS, 1), jnp.float32),
            pltpu.VMEM((N_EXPERTS, 1), jnp.float32),
        ],
    )(x, W, expert_bias.reshape(N_EXPERTS, 1))
    return weights.T, indices.T, loss[0, 0]


# speedup vs baseline: 1.0344x; 1.0344x over previous
import functools

import jax
import jax.numpy as jnp
from jax.experimental import pallas as pl
from jax.experimental.pallas import tpu as pltpu

DIM = 768
N_EXPERTS = 64
TOP_K = 8
TOKENS = 32768
BLOCK = 4096
NCHUNK = 16


def _router_kernel(x_ref, w_ref, b_ref, w_out, i_out, loss_out,
                   counts_ref, probsum_ref):
    step = pl.program_id(0)
    nsteps = pl.num_programs(0)

    @pl.when(step == 0)
    def _init():
        counts_ref[...] = jnp.zeros_like(counts_ref)
        probsum_ref[...] = jnp.zeros_like(probsum_ref)

    lt = jax.lax.dot_general(
        w_ref[...], x_ref[...],
        dimension_numbers=(((0,), (1,)), ((), ())),
        preferred_element_type=jnp.float32) + b_ref[...]
    s = jax.nn.sigmoid(lt)
    d = jnp.sum(s, axis=0, keepdims=True) + 1e-6
    rd = 1.0 / d
    probsum_ref[...] += jax.lax.dot_general(
        s, rd, dimension_numbers=(((1,), (1,)), ((), ())),
        preferred_element_type=jnp.float32)

    csz = BLOCK // NCHUNK
    eidx = jax.lax.broadcasted_iota(jnp.int32, (N_EXPERTS, csz), 0)
    works = [s[:, c * csz:(c + 1) * csz] for c in range(NCHUNK)]
    ws = [[] for _ in range(NCHUNK)]
    idxs = [[] for _ in range(NCHUNK)]
    for _ in range(TOP_K):
        for c in range(NCHUNK):
            v, i = works[c], eidx
            half = N_EXPERTS // 2
            while half >= 1:
                va, vb = v[:half], v[half:]
                ia, ib = i[:half], i[half:]
                keep = va >= vb
                v = jnp.maximum(va, vb)
                i = jnp.where(keep, ia, ib)
                half //= 2
            first_b = eidx == i
            ws[c].append(v)
            idxs[c].append(i)
            works[c] = jnp.where(first_b, -1.0, works[c])

    counts = jnp.zeros((N_EXPERTS, 1), jnp.float32)
    for c in range(NCHUNK):
        neg = (works[c] < 0.0).astype(jnp.float32)
        counts = counts + jnp.sum(neg, axis=1, keepdims=True)
    counts_ref[...] += counts

    wstack = jnp.concatenate(
        [jnp.concatenate(ws[c], axis=0) for c in range(NCHUNK)],
        axis=1)
    wstack = wstack / (jnp.sum(wstack, axis=0, keepdims=True) + 1e-6 * d)
    istack = jnp.concatenate(
        [jnp.concatenate(idxs[c], axis=0) for c in range(NCHUNK)],
        axis=1)

    w_out[...] = wstack
    i_out[...] = istack

    @pl.when(step == nsteps - 1)
    def _finalize():
        scale = N_EXPERTS / (TOKENS * TOKENS)
        loss = scale * jnp.sum(counts_ref[...] * probsum_ref[...])
        loss_out[...] = jnp.full((1, 1), loss, dtype=jnp.float32)


@functools.partial(jax.jit, static_argnames=())
def kernel(x, W, expert_bias):
    grid = TOKENS // BLOCK
    weights, indices, loss = pl.pallas_call(
        _router_kernel,
        grid=(grid,),
        in_specs=[
            pl.BlockSpec((BLOCK, DIM), lambda i: (i, 0)),
            pl.BlockSpec((DIM, N_EXPERTS), lambda i: (0, 0)),
            pl.BlockSpec((N_EXPERTS, 1), lambda i: (0, 0)),
        ],
        out_specs=[
            pl.BlockSpec((TOP_K, BLOCK), lambda i: (0, i)),
            pl.BlockSpec((TOP_K, BLOCK), lambda i: (0, i)),
            pl.BlockSpec((1, 1), lambda i: (0, 0)),
        ],
        out_shape=[
            jax.ShapeDtypeStruct((TOP_K, TOKENS), jnp.float32),
            jax.ShapeDtypeStruct((TOP_K, TOKENS), jnp.int32),
            jax.ShapeDtypeStruct((1, 1), jnp.float32),
        ],
        scratch_shapes=[
            pltpu.VMEM((N_EXPERTS, 1), jnp.float32),
            pltpu.VMEM((N_EXPERTS, 1), jnp.float32),
        ],
    )(x, W, expert_bias.reshape(N_EXPERTS, 1))
    return weights.T, indices.T, loss[0, 0]
